# Initial kernel scaffold; baseline (speedup 1.0000x reference)
import jax
import jax.numpy as jnp
from jax.experimental import pallas as pl


def kernel(source_features, source_labels, target_features, target_features_0, target_labels):
    raise NotImplementedError("write your pallas kernel here")


# speedup vs baseline: 73.2013x; 73.2013x over previous
import functools

import jax
import jax.numpy as jnp
from jax.experimental import pallas as pl

RANKING_K = 20
TOP_RANKED_N = 512
TOP_N_SIM = 5
TAU = 0.05
N_SRC = 4096
N_TGT = 2048
D = 256
BCOL = 256
NB = N_TGT // BCOL
EPS = 1e-12
BIGI = 1 << 30


def _normalize(x):
    n = jnp.sqrt(jnp.sum(x * x, axis=1, keepdims=True))
    return x / jnp.maximum(n, EPS)


def _extract_next(sim, elig, iota_r, val):
    m = jnp.max(jnp.where(elig, sim, -jnp.inf), axis=0, keepdims=True)
    cand = elig & (sim == m)
    idx = jnp.min(jnp.where(cand, iota_r, BIGI), axis=0, keepdims=True)
    chosen = iota_r == idx
    v = jnp.sum(jnp.where(chosen, val, jnp.zeros_like(val)), axis=0, keepdims=True)
    return v, elig & jnp.logical_not(chosen)


def _phase1_kernel(s_ref, t_ref, t0_ref, slab_ref, tlab_ref,
                   r_ref, asg_ref, ncorr_ref):
    j = pl.program_id(0)
    s = _normalize(s_ref[...])
    t = _normalize(t_ref[...])
    t0 = _normalize(t0_ref[...])
    dn = (((1,), (1,)), ((), ()))
    sim = jax.lax.dot_general(s, t, dn, preferred_element_type=jnp.float32)
    sim0 = jax.lax.dot_general(s, t0, dn, preferred_element_type=jnp.float32)
    labs = slab_ref[...]
    labs_f = labs.astype(jnp.float32)
    iota_r = jax.lax.broadcasted_iota(jnp.int32, (N_SRC, BCOL), 0)

    elig = jnp.ones((N_SRC, BCOL), dtype=jnp.bool_)
    top_labs = []
    for _ in range(TOP_N_SIM):
        lab, elig = _extract_next(sim, elig, iota_r, labs_f)
        top_labs.append(lab)
    counts = []
    for a in range(TOP_N_SIM):
        c = jnp.zeros_like(top_labs[0])
        for b in range(TOP_N_SIM):
            c = c + (top_labs[a] == top_labs[b]).astype(jnp.float32)
        counts.append(c)
    maxc = functools.reduce(jnp.maximum, counts)
    assigned_f = functools.reduce(
        jnp.minimum,
        [jnp.where(counts[a] == maxc, top_labs[a], jnp.float32(1e9))
         for a in range(TOP_N_SIM)])
    assigned = assigned_f.astype(jnp.int32)

    tlab = tlab_ref[...].reshape(1, BCOL)
    ncorr_part = jnp.sum((assigned == tlab).astype(jnp.int32))

    pos = labs == assigned
    elig_p = pos
    elig_n = jnp.logical_not(pos)
    nln = jnp.zeros((1, BCOL), jnp.float32)
    nun = jnp.zeros((1, BCOL), jnp.float32)
    for _ in range(RANKING_K):
        vp, elig_p = _extract_next(sim, elig_p, iota_r, sim0)
        nln = nln + vp
        vn, elig_n = _extract_next(sim, elig_n, iota_r, sim0)
        nun = nun + vn

    r_ref[...] = (nln / nun).reshape(1, 1, BCOL)
    asg_ref[...] = assigned.reshape(1, 1, BCOL)

    @pl.when(j == 0)
    def _():
        ncorr_ref[...] = jnp.zeros((1, 1), jnp.int32)
    ncorr_ref[...] += ncorr_part


def _sortable(x):
    b = jax.lax.bitcast_convert_type(x, jnp.int32)
    return jnp.where(b >= 0, b, b ^ jnp.int32(0x7FFFFFFF))


def _phase2_kernel(s_ref, t0_ref, slab_ref, rfull_ref, rblk_ref, asg_ref,
                   loss_ref):
    j = pl.program_id(0)
    skey = _sortable(rfull_ref[...])

    int_min = jnp.int32(-2147483648)
    cnt_pos = jnp.sum((skey >= 0).astype(jnp.int32), keepdims=True).reshape(1, 1)
    prefix = jnp.where(cnt_pos >= TOP_RANKED_N,
                       jnp.zeros((1, 1), jnp.int32),
                       jnp.full((1, 1), int_min, jnp.int32))
    for b in range(30, -1, -1):
        cand = prefix + jnp.int32(1 << b)
        cnt = jnp.sum((skey >= cand).astype(jnp.int32), keepdims=True).reshape(1, 1)
        prefix = jnp.where(cnt >= TOP_RANKED_N, cand, prefix)
    t512 = prefix
    c_strict = jnp.sum((skey > t512).astype(jnp.int32), keepdims=True).reshape(1, 1)
    tie_full = skey == t512
    gcol = jax.lax.broadcasted_iota(jnp.int32, (1, N_TGT), 1)
    prior = jnp.sum((tie_full & (gcol < j * BCOL)).astype(jnp.int32),
                    keepdims=True).reshape(1, 1)

    skeyb = _sortable(rblk_ref[...].reshape(1, BCOL))
    strict_b = skeyb > t512
    tie_b = skeyb == t512
    lt = (jax.lax.broadcasted_iota(jnp.int32, (BCOL, BCOL), 0)
          <= jax.lax.broadcasted_iota(jnp.int32, (BCOL, BCOL), 1)
          ).astype(jnp.float32)
    cum_b = jax.lax.dot_general(tie_b.astype(jnp.float32), lt,
                                (((1,), (0,)), ((), ())),
                                preferred_element_type=jnp.float32
                                ).astype(jnp.int32)
    sel = strict_b | (tie_b & ((prior + cum_b) <= (TOP_RANKED_N - c_strict)))

    s = _normalize(s_ref[...])
    t0 = _normalize(t0_ref[...])
    sim0 = jax.lax.dot_general(s, t0, (((1,), (1,)), ((), ())),
                               preferred_element_type=jnp.float32)
    labs = slab_ref[...]
    asg = asg_ref[...].reshape(1, BCOL)
    mask = (labs == asg).astype(jnp.float32)
    z = sim0 / jnp.float32(TAU)
    m = jnp.max(z, axis=0, keepdims=True)
    e = jnp.exp(z - m)
    den = jnp.sum(e, axis=0, keepdims=True)
    num = jnp.sum(e * mask, axis=0, keepdims=True)
    term = jnp.log(num / den + jnp.float32(1e-6))
    part = jnp.sum(jnp.where(sel, term, jnp.zeros_like(term)))

    @pl.when(j == 0)
    def _():
        loss_ref[...] = jnp.zeros((1, 1), jnp.float32)
    loss_ref[...] += part

    @pl.when(j == NB - 1)
    def _():
        loss_ref[...] = jnp.float32(-1.0) * (loss_ref[...] / jnp.float32(TOP_RANKED_N))


def kernel(source_features, source_labels, target_features, target_features_0,
           target_labels):
    slab2 = source_labels.reshape(N_SRC, 1).astype(jnp.int32)
    tlab3 = target_labels.reshape(NB, 1, BCOL).astype(jnp.int32)

    r3, asg3, ncorr = pl.pallas_call(
        _phase1_kernel,
        grid=(NB,),
        in_specs=[
            pl.BlockSpec((N_SRC, D), lambda j: (0, 0)),
            pl.BlockSpec((BCOL, D), lambda j: (j, 0)),
            pl.BlockSpec((BCOL, D), lambda j: (j, 0)),
            pl.BlockSpec((N_SRC, 1), lambda j: (0, 0)),
            pl.BlockSpec((1, 1, BCOL), lambda j: (j, 0, 0)),
        ],
        out_specs=[
            pl.BlockSpec((1, 1, BCOL), lambda j: (j, 0, 0)),
            pl.BlockSpec((1, 1, BCOL), lambda j: (j, 0, 0)),
            pl.BlockSpec((1, 1), lambda j: (0, 0)),
        ],
        out_shape=[
            jax.ShapeDtypeStruct((NB, 1, BCOL), jnp.float32),
            jax.ShapeDtypeStruct((NB, 1, BCOL), jnp.int32),
            jax.ShapeDtypeStruct((1, 1), jnp.int32),
        ],
    )(source_features, target_features, target_features_0, slab2, tlab3)

    rfull = r3.reshape(1, N_TGT)
    loss = pl.pallas_call(
        _phase2_kernel,
        grid=(NB,),
        in_specs=[
            pl.BlockSpec((N_SRC, D), lambda j: (0, 0)),
            pl.BlockSpec((BCOL, D), lambda j: (j, 0)),
            pl.BlockSpec((N_SRC, 1), lambda j: (0, 0)),
            pl.BlockSpec((1, N_TGT), lambda j: (0, 0)),
            pl.BlockSpec((1, 1, BCOL), lambda j: (j, 0, 0)),
            pl.BlockSpec((1, 1, BCOL), lambda j: (j, 0, 0)),
        ],
        out_specs=pl.BlockSpec((1, 1), lambda j: (0, 0)),
        out_shape=jax.ShapeDtypeStruct((1, 1), jnp.float32),
    )(source_features, target_features_0, slab2, rfull, r3, asg3)

    return loss.reshape(()), ncorr.reshape(()).astype(jnp.int32)


# speedup vs baseline: 177.5607x; 2.4256x over previous
import functools

import jax
import jax.numpy as jnp
from jax.experimental import pallas as pl
from jax.experimental.pallas import tpu as pltpu

RANKING_K = 20
TOP_RANKED_N = 512
TOP_N_SIM = 5
TAU = 0.05
N_SRC = 4096
N_TGT = 2048
D = 256
BCOL = 256
NB = N_TGT // BCOL
EPS = 1e-12
BIGI = 1 << 30


def _normalize(x):
    n = jnp.sqrt(jnp.sum(x * x, axis=1, keepdims=True))
    return x / jnp.maximum(n, EPS)


def _pop_max(mref, val):
    a = mref[...]
    m = jnp.max(a, axis=0, keepdims=True)
    mval = jnp.where(m == -jnp.inf, jnp.inf, m)
    cand = a == mval
    v = jnp.sum(jnp.where(cand, val, jnp.zeros_like(val)), axis=0, keepdims=True)
    mref[...] = jnp.where(cand, -jnp.inf, a)
    return v


def _phase1_kernel(s_ref, t_ref, t0_ref, slab_ref, tlab_ref,
                   r_ref, asg_ref, ncorr_ref, mA, mB):
    j = pl.program_id(0)
    s = _normalize(s_ref[...])
    t = _normalize(t_ref[...])
    t0 = _normalize(t0_ref[...])
    dn = (((1,), (1,)), ((), ()))
    sim = jax.lax.dot_general(s, t, dn, preferred_element_type=jnp.float32)
    sim0 = jax.lax.dot_general(s, t0, dn, preferred_element_type=jnp.float32)
    labs = slab_ref[...]
    labs_f = labs.astype(jnp.float32)

    mA[...] = sim
    top_labs = []
    for _ in range(TOP_N_SIM):
        top_labs.append(_pop_max(mA, labs_f))
    counts = []
    for a in range(TOP_N_SIM):
        c = jnp.zeros_like(top_labs[0])
        for b in range(TOP_N_SIM):
            c = c + (top_labs[a] == top_labs[b]).astype(jnp.float32)
        counts.append(c)
    maxc = functools.reduce(jnp.maximum, counts)
    assigned_f = functools.reduce(
        jnp.minimum,
        [jnp.where(counts[a] == maxc, top_labs[a], jnp.float32(1e9))
         for a in range(TOP_N_SIM)])
    assigned = assigned_f.astype(jnp.int32)

    tlab = tlab_ref[...].reshape(1, BCOL)
    ncorr_part = jnp.sum((assigned == tlab).astype(jnp.int32))

    pos = labs == assigned
    mA[...] = jnp.where(pos, sim, -jnp.inf)
    mB[...] = jnp.where(pos, -jnp.inf, sim)

    def body(_, carry):
        nln, nun = carry
        nln = nln + _pop_max(mA, sim0)
        nun = nun + _pop_max(mB, sim0)
        return nln, nun

    nln, nun = jax.lax.fori_loop(
        0, RANKING_K, body,
        (jnp.zeros((1, BCOL), jnp.float32), jnp.zeros((1, BCOL), jnp.float32)))

    r_ref[...] = (nln / nun).reshape(1, 1, BCOL)
    asg_ref[...] = assigned.reshape(1, 1, BCOL)

    @pl.when(j == 0)
    def _():
        ncorr_ref[...] = jnp.zeros((1, 1), jnp.int32)
    ncorr_ref[...] += ncorr_part


def _sortable(x):
    b = jax.lax.bitcast_convert_type(x, jnp.int32)
    return jnp.where(b >= 0, b, b ^ jnp.int32(0x7FFFFFFF))


def _phase2_kernel(s_ref, t0_ref, slab_ref, rfull_ref, rblk_ref, asg_ref,
                   loss_ref):
    j = pl.program_id(0)
    skey = _sortable(rfull_ref[...])

    int_min = jnp.int32(-2147483648)
    cnt_pos = jnp.sum((skey >= 0).astype(jnp.int32), keepdims=True).reshape(1, 1)
    prefix = jnp.where(cnt_pos >= TOP_RANKED_N,
                       jnp.zeros((1, 1), jnp.int32),
                       jnp.full((1, 1), int_min, jnp.int32))
    for b in range(30, -1, -1):
        cand = prefix + jnp.int32(1 << b)
        cnt = jnp.sum((skey >= cand).astype(jnp.int32), keepdims=True).reshape(1, 1)
        prefix = jnp.where(cnt >= TOP_RANKED_N, cand, prefix)
    t512 = prefix
    c_strict = jnp.sum((skey > t512).astype(jnp.int32), keepdims=True).reshape(1, 1)
    tie_full = skey == t512
    gcol = jax.lax.broadcasted_iota(jnp.int32, (1, N_TGT), 1)
    prior = jnp.sum((tie_full & (gcol < j * BCOL)).astype(jnp.int32),
                    keepdims=True).reshape(1, 1)

    skeyb = _sortable(rblk_ref[...].reshape(1, BCOL))
    strict_b = skeyb > t512
    tie_b = skeyb == t512
    lt = (jax.lax.broadcasted_iota(jnp.int32, (BCOL, BCOL), 0)
          <= jax.lax.broadcasted_iota(jnp.int32, (BCOL, BCOL), 1)
          ).astype(jnp.float32)
    cum_b = jax.lax.dot_general(tie_b.astype(jnp.float32), lt,
                                (((1,), (0,)), ((), ())),
                                preferred_element_type=jnp.float32
                                ).astype(jnp.int32)
    sel = strict_b | (tie_b & ((prior + cum_b) <= (TOP_RANKED_N - c_strict)))

    s = _normalize(s_ref[...])
    t0 = _normalize(t0_ref[...])
    sim0 = jax.lax.dot_general(s, t0, (((1,), (1,)), ((), ())),
                               preferred_element_type=jnp.float32)
    labs = slab_ref[...]
    asg = asg_ref[...].reshape(1, BCOL)
    mask = (labs == asg).astype(jnp.float32)
    z = sim0 / jnp.float32(TAU)
    m = jnp.max(z, axis=0, keepdims=True)
    e = jnp.exp(z - m)
    den = jnp.sum(e, axis=0, keepdims=True)
    num = jnp.sum(e * mask, axis=0, keepdims=True)
    term = jnp.log(num / den + jnp.float32(1e-6))
    part = jnp.sum(jnp.where(sel, term, jnp.zeros_like(term)))

    @pl.when(j == 0)
    def _():
        loss_ref[...] = jnp.zeros((1, 1), jnp.float32)
    loss_ref[...] += part

    @pl.when(j == NB - 1)
    def _():
        loss_ref[...] = jnp.float32(-1.0) * (loss_ref[...] / jnp.float32(TOP_RANKED_N))


def kernel(source_features, source_labels, target_features, target_features_0,
           target_labels):
    slab2 = source_labels.reshape(N_SRC, 1).astype(jnp.int32)
    tlab3 = target_labels.reshape(NB, 1, BCOL).astype(jnp.int32)

    r3, asg3, ncorr = pl.pallas_call(
        _phase1_kernel,
        grid=(NB,),
        in_specs=[
            pl.BlockSpec((N_SRC, D), lambda j: (0, 0)),
            pl.BlockSpec((BCOL, D), lambda j: (j, 0)),
            pl.BlockSpec((BCOL, D), lambda j: (j, 0)),
            pl.BlockSpec((N_SRC, 1), lambda j: (0, 0)),
            pl.BlockSpec((1, 1, BCOL), lambda j: (j, 0, 0)),
        ],
        out_specs=[
            pl.BlockSpec((1, 1, BCOL), lambda j: (j, 0, 0)),
            pl.BlockSpec((1, 1, BCOL), lambda j: (j, 0, 0)),
            pl.BlockSpec((1, 1), lambda j: (0, 0)),
        ],
        out_shape=[
            jax.ShapeDtypeStruct((NB, 1, BCOL), jnp.float32),
            jax.ShapeDtypeStruct((NB, 1, BCOL), jnp.int32),
            jax.ShapeDtypeStruct((1, 1), jnp.int32),
        ],
        scratch_shapes=[
            pltpu.VMEM((N_SRC, BCOL), jnp.float32),
            pltpu.VMEM((N_SRC, BCOL), jnp.float32),
        ],
    )(source_features, target_features, target_features_0, slab2, tlab3)

    rfull = r3.reshape(1, N_TGT)
    loss = pl.pallas_call(
        _phase2_kernel,
        grid=(NB,),
        in_specs=[
            pl.BlockSpec((N_SRC, D), lambda j: (0, 0)),
            pl.BlockSpec((BCOL, D), lambda j: (j, 0)),
            pl.BlockSpec((N_SRC, 1), lambda j: (0, 0)),
            pl.BlockSpec((1, N_TGT), lambda j: (0, 0)),
            pl.BlockSpec((1, 1, BCOL), lambda j: (j, 0, 0)),
            pl.BlockSpec((1, 1, BCOL), lambda j: (j, 0, 0)),
        ],
        out_specs=pl.BlockSpec((1, 1), lambda j: (0, 0)),
        out_shape=jax.ShapeDtypeStruct((1, 1), jnp.float32),
    )(source_features, target_features_0, slab2, rfull, r3, asg3)

    return loss.reshape(()), ncorr.reshape(()).astype(jnp.int32)


# speedup vs baseline: 235.5445x; 1.3266x over previous
import functools

import jax
import jax.numpy as jnp
from jax.experimental import pallas as pl
from jax.experimental.pallas import tpu as pltpu

RANKING_K = 20
TOP_RANKED_N = 512
TOP_N_SIM = 5
TAU = 0.05
N_SRC = 4096
N_TGT = 2048
D = 256
BCOL = 256
NB = N_TGT // BCOL
EPS = 1e-12
BIGI = 1 << 30


def _normalize(x):
    n = jnp.sqrt(jnp.sum(x * x, axis=1, keepdims=True))
    return x / jnp.maximum(n, EPS)


def _pop_max(mref, val):
    a = mref[...]
    m = jnp.max(a, axis=0, keepdims=True)
    mval = jnp.where(m == -jnp.inf, jnp.inf, m)
    cand = a == mval
    v = jnp.sum(jnp.where(cand, val, jnp.zeros_like(val)), axis=0, keepdims=True)
    mref[...] = jnp.where(cand, -jnp.inf, a)
    return v


def _phase1_kernel(s_ref, t_ref, t0_ref, slab_ref, tlab_ref,
                   r_ref, asg_ref, ncorr_ref, mA):
    j = pl.program_id(0)
    s = _normalize(s_ref[...])
    t = _normalize(t_ref[...])
    t0 = _normalize(t0_ref[...])
    dn = (((1,), (1,)), ((), ()))
    sim = jax.lax.dot_general(s, t, dn, preferred_element_type=jnp.float32)
    sim0 = jax.lax.dot_general(s, t0, dn, preferred_element_type=jnp.float32)
    labs = slab_ref[...]
    labs_f = labs.astype(jnp.float32)

    mA[...] = sim
    top_labs = []
    for _ in range(TOP_N_SIM):
        top_labs.append(_pop_max(mA, labs_f))
    counts = []
    for a in range(TOP_N_SIM):
        c = jnp.zeros_like(top_labs[0])
        for b in range(TOP_N_SIM):
            c = c + (top_labs[a] == top_labs[b]).astype(jnp.float32)
        counts.append(c)
    maxc = functools.reduce(jnp.maximum, counts)
    assigned_f = functools.reduce(
        jnp.minimum,
        [jnp.where(counts[a] == maxc, top_labs[a], jnp.float32(1e9))
         for a in range(TOP_N_SIM)])
    assigned = assigned_f.astype(jnp.int32)

    tlab = tlab_ref[...].reshape(1, BCOL)
    ncorr_part = jnp.sum((assigned == tlab).astype(jnp.int32))

    pos = labs == assigned
    int_min = jnp.int32(-2147483648)
    skey = _sortable(sim)
    kp = jnp.where(pos, skey, int_min)
    kn = jnp.where(pos, int_min, skey)

    def init_prefix(keys):
        cnt0 = jnp.sum((keys >= 0).astype(jnp.int32), axis=0, keepdims=True)
        return jnp.where(cnt0 >= RANKING_K,
                         jnp.zeros((1, BCOL), jnp.int32),
                         jnp.full((1, BCOL), int_min, jnp.int32))

    def bit_body(i, carry):
        pp, pn = carry
        bit = jnp.int32(1) << (jnp.int32(30) - i)
        candp = pp + bit
        candn = pn + bit
        cntp = jnp.sum((kp >= candp).astype(jnp.int32), axis=0, keepdims=True)
        cntn = jnp.sum((kn >= candn).astype(jnp.int32), axis=0, keepdims=True)
        return (jnp.where(cntp >= RANKING_K, candp, pp),
                jnp.where(cntn >= RANKING_K, candn, pn))

    tp, tn = jax.lax.fori_loop(0, 31, bit_body,
                               (init_prefix(kp), init_prefix(kn)))
    tpx = jnp.maximum(tp, int_min + 1)
    tnx = jnp.maximum(tn, int_min + 1)
    zero = jnp.zeros_like(sim0)
    nln = jnp.sum(jnp.where(kp >= tpx, sim0, zero), axis=0, keepdims=True)
    nun = jnp.sum(jnp.where(kn >= tnx, sim0, zero), axis=0, keepdims=True)

    r_ref[...] = (nln / nun).reshape(1, 1, BCOL)
    asg_ref[...] = assigned.reshape(1, 1, BCOL)

    @pl.when(j == 0)
    def _():
        ncorr_ref[...] = jnp.zeros((1, 1), jnp.int32)
    ncorr_ref[...] += ncorr_part


def _sortable(x):
    b = jax.lax.bitcast_convert_type(x, jnp.int32)
    return jnp.where(b >= 0, b, b ^ jnp.int32(0x7FFFFFFF))


def _phase2_kernel(s_ref, t0_ref, slab_ref, rfull_ref, rblk_ref, asg_ref,
                   loss_ref):
    j = pl.program_id(0)
    skey = _sortable(rfull_ref[...])

    int_min = jnp.int32(-2147483648)
    cnt_pos = jnp.sum((skey >= 0).astype(jnp.int32), keepdims=True).reshape(1, 1)
    prefix = jnp.where(cnt_pos >= TOP_RANKED_N,
                       jnp.zeros((1, 1), jnp.int32),
                       jnp.full((1, 1), int_min, jnp.int32))
    for b in range(30, -1, -1):
        cand = prefix + jnp.int32(1 << b)
        cnt = jnp.sum((skey >= cand).astype(jnp.int32), keepdims=True).reshape(1, 1)
        prefix = jnp.where(cnt >= TOP_RANKED_N, cand, prefix)
    t512 = prefix
    c_strict = jnp.sum((skey > t512).astype(jnp.int32), keepdims=True).reshape(1, 1)
    tie_full = skey == t512
    gcol = jax.lax.broadcasted_iota(jnp.int32, (1, N_TGT), 1)
    prior = jnp.sum((tie_full & (gcol < j * BCOL)).astype(jnp.int32),
                    keepdims=True).reshape(1, 1)

    skeyb = _sortable(rblk_ref[...].reshape(1, BCOL))
    strict_b = skeyb > t512
    tie_b = skeyb == t512
    lt = (jax.lax.broadcasted_iota(jnp.int32, (BCOL, BCOL), 0)
          <= jax.lax.broadcasted_iota(jnp.int32, (BCOL, BCOL), 1)
          ).astype(jnp.float32)
    cum_b = jax.lax.dot_general(tie_b.astype(jnp.float32), lt,
                                (((1,), (0,)), ((), ())),
                                preferred_element_type=jnp.float32
                                ).astype(jnp.int32)
    sel = strict_b | (tie_b & ((prior + cum_b) <= (TOP_RANKED_N - c_strict)))

    s = _normalize(s_ref[...])
    t0 = _normalize(t0_ref[...])
    sim0 = jax.lax.dot_general(s, t0, (((1,), (1,)), ((), ())),
                               preferred_element_type=jnp.float32)
    labs = slab_ref[...]
    asg = asg_ref[...].reshape(1, BCOL)
    mask = (labs == asg).astype(jnp.float32)
    z = sim0 / jnp.float32(TAU)
    m = jnp.max(z, axis=0, keepdims=True)
    e = jnp.exp(z - m)
    den = jnp.sum(e, axis=0, keepdims=True)
    num = jnp.sum(e * mask, axis=0, keepdims=True)
    term = jnp.log(num / den + jnp.float32(1e-6))
    part = jnp.sum(jnp.where(sel, term, jnp.zeros_like(term)))

    @pl.when(j == 0)
    def _():
        loss_ref[...] = jnp.zeros((1, 1), jnp.float32)
    loss_ref[...] += part

    @pl.when(j == NB - 1)
    def _():
        loss_ref[...] = jnp.float32(-1.0) * (loss_ref[...] / jnp.float32(TOP_RANKED_N))


def kernel(source_features, source_labels, target_features, target_features_0,
           target_labels):
    slab2 = source_labels.reshape(N_SRC, 1).astype(jnp.int32)
    tlab3 = target_labels.reshape(NB, 1, BCOL).astype(jnp.int32)

    r3, asg3, ncorr = pl.pallas_call(
        _phase1_kernel,
        grid=(NB,),
        in_specs=[
            pl.BlockSpec((N_SRC, D), lambda j: (0, 0)),
            pl.BlockSpec((BCOL, D), lambda j: (j, 0)),
            pl.BlockSpec((BCOL, D), lambda j: (j, 0)),
            pl.BlockSpec((N_SRC, 1), lambda j: (0, 0)),
            pl.BlockSpec((1, 1, BCOL), lambda j: (j, 0, 0)),
        ],
        out_specs=[
            pl.BlockSpec((1, 1, BCOL), lambda j: (j, 0, 0)),
            pl.BlockSpec((1, 1, BCOL), lambda j: (j, 0, 0)),
            pl.BlockSpec((1, 1), lambda j: (0, 0)),
        ],
        out_shape=[
            jax.ShapeDtypeStruct((NB, 1, BCOL), jnp.float32),
            jax.ShapeDtypeStruct((NB, 1, BCOL), jnp.int32),
            jax.ShapeDtypeStruct((1, 1), jnp.int32),
        ],
        scratch_shapes=[
            pltpu.VMEM((N_SRC, BCOL), jnp.float32),
        ],
    )(source_features, target_features, target_features_0, slab2, tlab3)

    rfull = r3.reshape(1, N_TGT)
    loss = pl.pallas_call(
        _phase2_kernel,
        grid=(NB,),
        in_specs=[
            pl.BlockSpec((N_SRC, D), lambda j: (0, 0)),
            pl.BlockSpec((BCOL, D), lambda j: (j, 0)),
            pl.BlockSpec((N_SRC, 1), lambda j: (0, 0)),
            pl.BlockSpec((1, N_TGT), lambda j: (0, 0)),
            pl.BlockSpec((1, 1, BCOL), lambda j: (j, 0, 0)),
            pl.BlockSpec((1, 1, BCOL), lambda j: (j, 0, 0)),
        ],
        out_specs=pl.BlockSpec((1, 1), lambda j: (0, 0)),
        out_shape=jax.ShapeDtypeStruct((1, 1), jnp.float32),
    )(source_features, target_features_0, slab2, rfull, r3, asg3)

    return loss.reshape(()), ncorr.reshape(()).astype(jnp.int32)


# speedup vs baseline: 272.7370x; 1.1579x over previous
import functools

import jax
import jax.numpy as jnp
from jax.experimental import pallas as pl
from jax.experimental.pallas import tpu as pltpu

RANKING_K = 20
TOP_RANKED_N = 512
TOP_N_SIM = 5
TAU = 0.05
N_SRC = 4096
N_TGT = 2048
D = 256
BCOL = 256
NB = N_TGT // BCOL
EPS = 1e-12
BIGI = 1 << 30


def _normalize(x):
    n = jnp.sqrt(jnp.sum(x * x, axis=1, keepdims=True))
    return x / jnp.maximum(n, EPS)


def _pop_max(mref, val):
    a = mref[...]
    m = jnp.max(a, axis=0, keepdims=True)
    mval = jnp.where(m == -jnp.inf, jnp.inf, m)
    cand = a == mval
    v = jnp.sum(jnp.where(cand, val, jnp.zeros_like(val)), axis=0, keepdims=True)
    mref[...] = jnp.where(cand, -jnp.inf, a)
    return v


def _phase1_kernel(s_ref, t_ref, t0_ref, slab_ref, tlab_ref,
                   r_ref, asg_ref, ncorr_ref, mA):
    j = pl.program_id(0)
    s = _normalize(s_ref[...])
    t = _normalize(t_ref[...])
    t0 = _normalize(t0_ref[...])
    dn = (((1,), (1,)), ((), ()))
    sim = jax.lax.dot_general(s, t, dn, preferred_element_type=jnp.float32)
    sim0 = jax.lax.dot_general(s, t0, dn, preferred_element_type=jnp.float32)
    labs = slab_ref[...]
    labs_f = labs.astype(jnp.float32)

    mA[...] = sim
    top_labs = []
    for _ in range(TOP_N_SIM):
        top_labs.append(_pop_max(mA, labs_f))
    counts = []
    for a in range(TOP_N_SIM):
        c = jnp.zeros_like(top_labs[0])
        for b in range(TOP_N_SIM):
            c = c + (top_labs[a] == top_labs[b]).astype(jnp.float32)
        counts.append(c)
    maxc = functools.reduce(jnp.maximum, counts)
    assigned_f = functools.reduce(
        jnp.minimum,
        [jnp.where(counts[a] == maxc, top_labs[a], jnp.float32(1e9))
         for a in range(TOP_N_SIM)])
    assigned = assigned_f.astype(jnp.int32)

    tlab = tlab_ref[...].reshape(1, BCOL)
    ncorr_part = jnp.sum((assigned == tlab).astype(jnp.int32))

    pos = labs == assigned
    int_min = jnp.int32(-2147483648)
    skey = _sortable(sim)
    kp = jnp.where(pos, skey, int_min)
    kn = jnp.where(pos, int_min, skey)

    ones_row = jnp.ones((1, N_SRC), jnp.float32)
    kf = jnp.float32(RANKING_K)

    def count_ge(keys, cand):
        ge = jnp.where(keys >= cand, jnp.float32(1.0), jnp.float32(0.0))
        return jax.lax.dot_general(ones_row, ge, (((1,), (0,)), ((), ())),
                                   preferred_element_type=jnp.float32)

    def init_prefix(keys):
        cnt0 = count_ge(keys, jnp.zeros((1, BCOL), jnp.int32))
        return jnp.where(cnt0 >= kf,
                         jnp.zeros((1, BCOL), jnp.int32),
                         jnp.full((1, BCOL), int_min, jnp.int32))

    def bit_body(i, carry):
        pp, pn = carry
        bit = jnp.int32(1) << (jnp.int32(30) - i)
        candp = pp + bit
        candn = pn + bit
        cntp = count_ge(kp, candp)
        cntn = count_ge(kn, candn)
        return (jnp.where(cntp >= kf, candp, pp),
                jnp.where(cntn >= kf, candn, pn))

    tp, tn = jax.lax.fori_loop(0, 31, bit_body,
                               (init_prefix(kp), init_prefix(kn)))
    tpx = jnp.maximum(tp, int_min + 1)
    tnx = jnp.maximum(tn, int_min + 1)
    zero = jnp.zeros_like(sim0)
    nln = jnp.sum(jnp.where(kp >= tpx, sim0, zero), axis=0, keepdims=True)
    nun = jnp.sum(jnp.where(kn >= tnx, sim0, zero), axis=0, keepdims=True)

    r_ref[...] = (nln / nun).reshape(1, 1, BCOL)
    asg_ref[...] = assigned.reshape(1, 1, BCOL)

    @pl.when(j == 0)
    def _():
        ncorr_ref[...] = jnp.zeros((1, 1), jnp.int32)
    ncorr_ref[...] += ncorr_part


def _sortable(x):
    b = jax.lax.bitcast_convert_type(x, jnp.int32)
    return jnp.where(b >= 0, b, b ^ jnp.int32(0x7FFFFFFF))


def _phase2_kernel(s_ref, t0_ref, slab_ref, rfull_ref, rblk_ref, asg_ref,
                   loss_ref):
    j = pl.program_id(0)
    skey = _sortable(rfull_ref[...])

    int_min = jnp.int32(-2147483648)
    cnt_pos = jnp.sum((skey >= 0).astype(jnp.int32), keepdims=True).reshape(1, 1)
    prefix = jnp.where(cnt_pos >= TOP_RANKED_N,
                       jnp.zeros((1, 1), jnp.int32),
                       jnp.full((1, 1), int_min, jnp.int32))
    for b in range(30, -1, -1):
        cand = prefix + jnp.int32(1 << b)
        cnt = jnp.sum((skey >= cand).astype(jnp.int32), keepdims=True).reshape(1, 1)
        prefix = jnp.where(cnt >= TOP_RANKED_N, cand, prefix)
    t512 = prefix
    c_strict = jnp.sum((skey > t512).astype(jnp.int32), keepdims=True).reshape(1, 1)
    tie_full = skey == t512
    gcol = jax.lax.broadcasted_iota(jnp.int32, (1, N_TGT), 1)
    prior = jnp.sum((tie_full & (gcol < j * BCOL)).astype(jnp.int32),
                    keepdims=True).reshape(1, 1)

    skeyb = _sortable(rblk_ref[...].reshape(1, BCOL))
    strict_b = skeyb > t512
    tie_b = skeyb == t512
    lt = (jax.lax.broadcasted_iota(jnp.int32, (BCOL, BCOL), 0)
          <= jax.lax.broadcasted_iota(jnp.int32, (BCOL, BCOL), 1)
          ).astype(jnp.float32)
    cum_b = jax.lax.dot_general(tie_b.astype(jnp.float32), lt,
                                (((1,), (0,)), ((), ())),
                                preferred_element_type=jnp.float32
                                ).astype(jnp.int32)
    sel = strict_b | (tie_b & ((prior + cum_b) <= (TOP_RANKED_N - c_strict)))

    s = _normalize(s_ref[...])
    t0 = _normalize(t0_ref[...])
    sim0 = jax.lax.dot_general(s, t0, (((1,), (1,)), ((), ())),
                               preferred_element_type=jnp.float32)
    labs = slab_ref[...]
    asg = asg_ref[...].reshape(1, BCOL)
    mask = (labs == asg).astype(jnp.float32)
    z = sim0 / jnp.float32(TAU)
    m = jnp.max(z, axis=0, keepdims=True)
    e = jnp.exp(z - m)
    den = jnp.sum(e, axis=0, keepdims=True)
    num = jnp.sum(e * mask, axis=0, keepdims=True)
    term = jnp.log(num / den + jnp.float32(1e-6))
    part = jnp.sum(jnp.where(sel, term, jnp.zeros_like(term)))

    @pl.when(j == 0)
    def _():
        loss_ref[...] = jnp.zeros((1, 1), jnp.float32)
    loss_ref[...] += part

    @pl.when(j == NB - 1)
    def _():
        loss_ref[...] = jnp.float32(-1.0) * (loss_ref[...] / jnp.float32(TOP_RANKED_N))


def kernel(source_features, source_labels, target_features, target_features_0,
           target_labels):
    slab2 = source_labels.reshape(N_SRC, 1).astype(jnp.int32)
    tlab3 = target_labels.reshape(NB, 1, BCOL).astype(jnp.int32)

    r3, asg3, ncorr = pl.pallas_call(
        _phase1_kernel,
        grid=(NB,),
        in_specs=[
            pl.BlockSpec((N_SRC, D), lambda j: (0, 0)),
            pl.BlockSpec((BCOL, D), lambda j: (j, 0)),
            pl.BlockSpec((BCOL, D), lambda j: (j, 0)),
            pl.BlockSpec((N_SRC, 1), lambda j: (0, 0)),
            pl.BlockSpec((1, 1, BCOL), lambda j: (j, 0, 0)),
        ],
        out_specs=[
            pl.BlockSpec((1, 1, BCOL), lambda j: (j, 0, 0)),
            pl.BlockSpec((1, 1, BCOL), lambda j: (j, 0, 0)),
            pl.BlockSpec((1, 1), lambda j: (0, 0)),
        ],
        out_shape=[
            jax.ShapeDtypeStruct((NB, 1, BCOL), jnp.float32),
            jax.ShapeDtypeStruct((NB, 1, BCOL), jnp.int32),
            jax.ShapeDtypeStruct((1, 1), jnp.int32),
        ],
        scratch_shapes=[
            pltpu.VMEM((N_SRC, BCOL), jnp.float32),
        ],
    )(source_features, target_features, target_features_0, slab2, tlab3)

    rfull = r3.reshape(1, N_TGT)
    loss = pl.pallas_call(
        _phase2_kernel,
        grid=(NB,),
        in_specs=[
            pl.BlockSpec((N_SRC, D), lambda j: (0, 0)),
            pl.BlockSpec((BCOL, D), lambda j: (j, 0)),
            pl.BlockSpec((N_SRC, 1), lambda j: (0, 0)),
            pl.BlockSpec((1, N_TGT), lambda j: (0, 0)),
            pl.BlockSpec((1, 1, BCOL), lambda j: (j, 0, 0)),
            pl.BlockSpec((1, 1, BCOL), lambda j: (j, 0, 0)),
        ],
        out_specs=pl.BlockSpec((1, 1), lambda j: (0, 0)),
        out_shape=jax.ShapeDtypeStruct((1, 1), jnp.float32),
    )(source_features, target_features_0, slab2, rfull, r3, asg3)

    return loss.reshape(()), ncorr.reshape(()).astype(jnp.int32)
